# baseline (speedup 1.0000x reference)
import functools
import numpy as np
import jax
import jax.numpy as jnp
from jax.experimental import pallas as pl
from jax.experimental.pallas import tpu as pltpu

SOFTNESS = 0.7

_KEY_HI = np.uint32(0)
_KEY_LO = np.uint32(42)
_KS2 = np.uint32(int(_KEY_HI) ^ int(_KEY_LO) ^ 0x1BD11BDA)

_TINY = np.float32(np.finfo(np.float32).tiny)
_SPAN = np.float32(np.float32(1.0) - _TINY)

_TS = 8
_LANES = 1024


def _rotl(x, d):
    return (x << np.uint32(d)) | (x >> np.uint32(32 - d))


def _threefry2x32_postkey(x0, x1):
    ks = (_KEY_HI, _KEY_LO, _KS2)
    rot0 = (13, 15, 26, 6)
    rot1 = (17, 29, 16, 24)

    def rounds(x0, x1, rots):
        for r in rots:
            x0 = x0 + x1
            x1 = _rotl(x1, r)
            x1 = x1 ^ x0
        return x0, x1

    x0, x1 = rounds(x0, x1, rot0)
    x0 = x0 + ks[1]; x1 = x1 + (ks[2] + np.uint32(1))
    x0, x1 = rounds(x0, x1, rot1)
    x0 = x0 + ks[2]; x1 = x1 + (ks[0] + np.uint32(2))
    x0, x1 = rounds(x0, x1, rot0)
    x0 = x0 + ks[0]; x1 = x1 + (ks[1] + np.uint32(3))
    x0, x1 = rounds(x0, x1, rot1)
    x0 = x0 + ks[1]; x1 = x1 + (ks[2] + np.uint32(4))
    x0, x1 = rounds(x0, x1, rot0)
    x0 = x0 + ks[2]; x1 = x1 + (ks[0] + np.uint32(5))
    return x0, x1


def _sample_body(l_ref, out_ref, *, n_tiles, sh_bn, sh_n):
    b = pl.program_id(0)
    s = pl.program_id(1)

    shape = (_TS, 8, 128)
    nvec = (jax.lax.broadcasted_iota(jnp.int32, shape, 0) + s * _TS).astype(jnp.uint32)
    sub = jax.lax.broadcasted_iota(jnp.int32, shape, 1).astype(jnp.uint32)
    lane = jax.lax.broadcasted_iota(jnp.int32, shape, 2).astype(jnp.uint32)
    hi = nvec >> np.uint32(32 - sh_bn)
    lo0 = (nvec << np.uint32(sh_bn)) + (b.astype(jnp.uint32) << np.uint32(sh_n)) \
        + sub * np.uint32(128) + lane
    kbase = (sub * np.uint32(128) + lane).astype(jnp.int32)
    neg_ln2 = np.float32(-np.log(np.float32(2.0)))

    def step(t, carry):
        best_v, best_t = carry
        x1 = lo0 + np.uint32(int(_KEY_LO) + t * _LANES)
        o0, o1 = _threefry2x32_postkey(hi, x1)
        bits = o0 ^ o1
        fb = (bits >> np.uint32(9)) | np.uint32(0x3F800000)
        u = jax.lax.bitcast_convert_type(fb, jnp.float32) - np.float32(1.0)
        y = jnp.log2(u) * neg_ln2
        g = jnp.log2(y) * neg_ln2
        lt = l_ref[0, t]
        v = g + lt[None, :, :]
        upd = v > best_v
        best_v = jnp.where(upd, v, best_v)
        best_t = jnp.where(upd, jnp.full(shape, t, jnp.int32), best_t)
        return best_v, best_t

    init_v = jnp.full(shape, -np.float32(3.0e38), jnp.float32)
    init_t = jnp.zeros(shape, jnp.int32)
    carry = (init_v, init_t)
    for t in range(n_tiles):
        carry = step(t, carry)
    best_v, best_t = carry

    best_k = best_t * np.int32(_LANES) + kbase
    vmax = jnp.max(jnp.max(best_v, axis=2, keepdims=True), axis=1, keepdims=True)
    cand = jnp.where(best_v == vmax, best_k, np.int32(2**31 - 1))
    kmin = jnp.min(jnp.min(cand, axis=2, keepdims=True), axis=1, keepdims=True)
    out_ref[0, 0] = jnp.broadcast_to(kmin, (_TS, 1, 128))


def _sample_idx(used_weight, interpret=False):
    B, N = used_weight.shape
    n_tiles = N // _LANES
    S = N
    sh_bn = int(np.log2(B * N))
    sh_n = int(np.log2(N))
    l4 = used_weight.reshape(B, n_tiles, 8, 128)
    grid = (B, S // _TS)
    out = pl.pallas_call(
        functools.partial(_sample_body, n_tiles=n_tiles, sh_bn=sh_bn, sh_n=sh_n),
        grid=grid,
        in_specs=[pl.BlockSpec((1, n_tiles, 8, 128), lambda b, s: (b, 0, 0, 0))],
        out_specs=pl.BlockSpec((1, 1, _TS, 1, 128),
                               lambda b, s: (b, s, 0, 0, 0)),
        out_shape=jax.ShapeDtypeStruct((B, S // _TS, _TS, 1, 128), jnp.int32),
        compiler_params=pltpu.CompilerParams(
            dimension_semantics=("parallel", "parallel")),
        interpret=interpret,
    )(l4)
    return out[:, :, :, 0, 0].reshape(B, S)


def kernel(state, weight):
    B, N = weight.shape
    log_n = jnp.log(jnp.asarray(N, dtype=jnp.float32))
    log_softness = jnp.log(jnp.asarray(SOFTNESS, dtype=jnp.float32))
    neg_log_softness = jnp.log(jnp.asarray(1.0 - SOFTNESS, dtype=jnp.float32))
    soft_weight = jnp.logaddexp(weight + log_softness, neg_log_softness - log_n)
    used_weight = soft_weight - jax.nn.logsumexp(soft_weight, axis=-1, keepdims=True)

    idx = _sample_idx(used_weight)

    new_state = jnp.take_along_axis(state, idx[:, :, None], axis=1)
    sel_w = jnp.take_along_axis(weight, idx, axis=1)
    sel_uw = jnp.take_along_axis(used_weight, idx, axis=1)
    new_weight = sel_w - sel_uw - log_n
    return (new_state, new_weight)


# speedup vs baseline: 1.0214x; 1.0214x over previous
import functools
import numpy as np
import jax
import jax.numpy as jnp
from jax.experimental import pallas as pl
from jax.experimental.pallas import tpu as pltpu

SOFTNESS = 0.7

_KEY_HI = np.uint32(0)
_KEY_LO = np.uint32(42)
_KS2 = np.uint32(int(_KEY_HI) ^ int(_KEY_LO) ^ 0x1BD11BDA)

_TINY = np.float32(np.finfo(np.float32).tiny)
_SPAN = np.float32(np.float32(1.0) - _TINY)

_TS = 8
_LANES = 1024


def _rotl(x, d):
    return (x << np.uint32(d)) | (x >> np.uint32(32 - d))


def _threefry2x32_postkey(x0, x1):
    ks = (_KEY_HI, _KEY_LO, _KS2)
    rot0 = (13, 15, 26, 6)
    rot1 = (17, 29, 16, 24)

    def rounds(x0, x1, rots):
        for r in rots:
            x0 = x0 + x1
            x1 = _rotl(x1, r)
            x1 = x1 ^ x0
        return x0, x1

    x0, x1 = rounds(x0, x1, rot0)
    x0 = x0 + ks[1]; x1 = x1 + (ks[2] + np.uint32(1))
    x0, x1 = rounds(x0, x1, rot1)
    x0 = x0 + ks[2]; x1 = x1 + (ks[0] + np.uint32(2))
    x0, x1 = rounds(x0, x1, rot0)
    x0 = x0 + ks[0]; x1 = x1 + (ks[1] + np.uint32(3))
    x0, x1 = rounds(x0, x1, rot1)
    x0 = x0 + ks[1]; x1 = x1 + (ks[2] + np.uint32(4))
    x0, x1 = rounds(x0, x1, rot0)
    x0 = x0 + ks[2]; x1 = x1 + (ks[0] + np.uint32(5))
    return x0, x1


def _sample_body(l_ref, out_ref, *, n_tiles, sh_bn, sh_n):
    b = pl.program_id(0)
    s = pl.program_id(1)

    shape = (_TS, 8, 128)
    nvec = (jax.lax.broadcasted_iota(jnp.int32, shape, 0) + s * _TS).astype(jnp.uint32)
    sub = jax.lax.broadcasted_iota(jnp.int32, shape, 1).astype(jnp.uint32)
    lane = jax.lax.broadcasted_iota(jnp.int32, shape, 2).astype(jnp.uint32)
    hi = nvec >> np.uint32(32 - sh_bn)
    lo0 = (nvec << np.uint32(sh_bn)) + (b.astype(jnp.uint32) << np.uint32(sh_n)) \
        + sub * np.uint32(128) + lane
    kbase = (sub * np.uint32(128) + lane).astype(jnp.int32)

    def step(t, carry):
        best_m, best_t = carry
        x1 = lo0 + np.uint32(int(_KEY_LO) + t * _LANES)
        o0, o1 = _threefry2x32_postkey(hi, x1)
        bits = o0 ^ o1
        fb = (bits >> np.uint32(9)) | np.uint32(0x3F800000)
        u = jax.lax.bitcast_convert_type(fb, jnp.float32) - np.float32(1.0)
        y = np.float32(0.0) - jnp.log(u)
        q = jnp.log(y)
        lt = l_ref[0, t]
        m = q - lt[None, :, :]
        upd = m < best_m
        best_m = jnp.where(upd, m, best_m)
        best_t = jnp.where(upd, jnp.full(shape, t, jnp.int32), best_t)
        return best_m, best_t

    init_m = jnp.full(shape, np.float32(3.0e38), jnp.float32)
    init_t = jnp.zeros(shape, jnp.int32)
    carry = (init_m, init_t)
    for t in range(n_tiles):
        carry = step(t, carry)
    best_m, best_t = carry

    best_k = best_t * np.int32(_LANES) + kbase
    mmin = jnp.min(jnp.min(best_m, axis=2, keepdims=True), axis=1, keepdims=True)
    cand = jnp.where(best_m == mmin, best_k, np.int32(2**31 - 1))
    kmin = jnp.min(jnp.min(cand, axis=2, keepdims=True), axis=1, keepdims=True)
    out_ref[0, 0] = jnp.broadcast_to(kmin, (_TS, 1, 128))


def _sample_idx(used_weight, interpret=False):
    B, N = used_weight.shape
    n_tiles = N // _LANES
    S = N
    sh_bn = int(np.log2(B * N))
    sh_n = int(np.log2(N))
    l4 = used_weight.reshape(B, n_tiles, 8, 128)
    grid = (B, S // _TS)
    out = pl.pallas_call(
        functools.partial(_sample_body, n_tiles=n_tiles, sh_bn=sh_bn, sh_n=sh_n),
        grid=grid,
        in_specs=[pl.BlockSpec((1, n_tiles, 8, 128), lambda b, s: (b, 0, 0, 0))],
        out_specs=pl.BlockSpec((1, 1, _TS, 1, 128),
                               lambda b, s: (b, s, 0, 0, 0)),
        out_shape=jax.ShapeDtypeStruct((B, S // _TS, _TS, 1, 128), jnp.int32),
        compiler_params=pltpu.CompilerParams(
            dimension_semantics=("parallel", "parallel")),
        interpret=interpret,
    )(l4)
    return out[:, :, :, 0, 0].reshape(B, S)


def kernel(state, weight):
    B, N = weight.shape
    log_n = jnp.log(jnp.asarray(N, dtype=jnp.float32))
    log_softness = jnp.log(jnp.asarray(SOFTNESS, dtype=jnp.float32))
    neg_log_softness = jnp.log(jnp.asarray(1.0 - SOFTNESS, dtype=jnp.float32))
    soft_weight = jnp.logaddexp(weight + log_softness, neg_log_softness - log_n)
    used_weight = soft_weight - jax.nn.logsumexp(soft_weight, axis=-1, keepdims=True)

    idx = _sample_idx(used_weight)

    new_state = jnp.take_along_axis(state, idx[:, :, None], axis=1)
    sel_w = jnp.take_along_axis(weight, idx, axis=1)
    sel_uw = jnp.take_along_axis(used_weight, idx, axis=1)
    new_weight = sel_w - sel_uw - log_n
    return (new_state, new_weight)
